# Initial kernel scaffold; baseline (speedup 1.0000x reference)
import jax
import jax.numpy as jnp
from jax.experimental import pallas as pl


def kernel(input_ids, emb_table, fc_w, fc_b):
    raise NotImplementedError("write your pallas kernel here")


# speedup vs baseline: 1.0323x; 1.0323x over previous
import functools

import jax
import jax.numpy as jnp
from jax import lax
from jax.experimental import pallas as pl
from jax.experimental.pallas import tpu as pltpu
from jax.experimental.pallas import tpu_sc as plsc

BATCH = 4096
SEQ = 200
EMBED_DIM = 64
NUM_CLASSES = 200

NUM_CORES = 2
NUM_SUBCORES = 16
NUM_WORKERS = NUM_CORES * NUM_SUBCORES
ROWS_PER_WORKER = BATCH // NUM_WORKERS
NBUF = 2
SPLIT_A = 104
SPLIT_B = SEQ - SPLIT_A
LANES = 16
DCHUNKS = EMBED_DIM // LANES


def _sc_body(ids_hbm, table_hbm, out_hbm, idx_v, bufs, aggbuf, t01, sems):
    wid = lax.axis_index("s") * NUM_CORES + lax.axis_index("c")
    base = wid * ROWS_PER_WORKER

    pltpu.sync_copy(ids_hbm.at[pl.ds(base, ROWS_PER_WORKER)], idx_v)
    pltpu.sync_copy(table_hbm.at[pl.ds(0, 2)], t01)

    def gather_row(r, buf, sem):
        c0 = pltpu.make_async_copy(
            table_hbm.at[idx_v.at[r, pl.ds(0, SPLIT_A)]],
            buf.at[pl.ds(0, SPLIT_A)],
            sem,
        )
        c1 = pltpu.make_async_copy(
            table_hbm.at[idx_v.at[r, pl.ds(SPLIT_A, SPLIT_B)]],
            buf.at[pl.ds(SPLIT_A, SPLIT_B)],
            sem,
        )
        return c0, c1

    for p in range(NBUF):
        c0, c1 = gather_row(p, bufs[p], sems[p])
        c0.start()
        c1.start()

    lane = lax.iota(jnp.int32, 16)
    tail_mask = lane >= (LANES - SEQ % LANES)

    def process_row(r, buf):
        n0 = jnp.zeros((LANES,), jnp.int32)
        n1 = jnp.zeros((LANES,), jnp.int32)
        for k in range(SEQ // LANES):
            v = idx_v[r, pl.ds(k * LANES, LANES)]
            n0 += plsc.all_reduce_population_count(v == 0)
            n1 += plsc.all_reduce_population_count(v == 1)
        v = idx_v[r, pl.ds(SEQ - LANES, LANES)]
        n0 += plsc.all_reduce_population_count((v == 0) & tail_mask)
        n1 += plsc.all_reduce_population_count((v == 1) & tail_mask)

        def acc_body(j, accs):
            return tuple(
                accs[c] + buf[j, pl.ds(c * LANES, LANES)] for c in range(DCHUNKS)
            )

        zero = jnp.zeros((LANES,), jnp.float32)
        accs = lax.fori_loop(0, SEQ, acc_body, (zero,) * DCHUNKS)

        n0f = n0.astype(jnp.float32)
        n1f = n1.astype(jnp.float32)
        inv = 1.0 / (float(SEQ) - n0f - n1f)
        for c in range(DCHUNKS):
            t0c = t01[0, pl.ds(c * LANES, LANES)]
            t1c = t01[1, pl.ds(c * LANES, LANES)]
            aggbuf[r, pl.ds(c * LANES, LANES)] = (
                accs[c] - n0f * t0c - n1f * t1c
            ) * inv

    def loop_body(i, carry):
        for p in range(NBUF):
            r = i * NBUF + p
            c0, c1 = gather_row(r, bufs[p], sems[p])
            c0.wait()
            c1.wait()
            process_row(r, bufs[p])

            @pl.when(r + NBUF < ROWS_PER_WORKER)
            def _():
                d0, d1 = gather_row(r + NBUF, bufs[p], sems[p])
                d0.start()
                d1.start()

        return carry

    lax.fori_loop(0, ROWS_PER_WORKER // NBUF, loop_body, 0)

    pltpu.sync_copy(aggbuf, out_hbm.at[pl.ds(base, ROWS_PER_WORKER)])


def _sc_agg(input_ids, emb_table):
    mesh = plsc.VectorSubcoreMesh(
        core_axis_name="c",
        subcore_axis_name="s",
        num_cores=NUM_CORES,
        num_subcores=NUM_SUBCORES,
    )

    def body(ids_hbm, table_hbm, out_hbm, idx_v, b0, b1, aggbuf, t01, s0, s1):
        _sc_body(
            ids_hbm, table_hbm, out_hbm, idx_v, (b0, b1), aggbuf, t01, (s0, s1)
        )

    return pl.kernel(
        body,
        out_type=jax.ShapeDtypeStruct((BATCH, EMBED_DIM), jnp.float32),
        mesh=mesh,
        compiler_params=pltpu.CompilerParams(
            use_tc_tiling_on_sc=False, needs_layout_passes=False
        ),
        scratch_types=[
            pltpu.VMEM((ROWS_PER_WORKER, SEQ), jnp.int32),
            pltpu.VMEM((SEQ, EMBED_DIM), jnp.float32),
            pltpu.VMEM((SEQ, EMBED_DIM), jnp.float32),
            pltpu.VMEM((ROWS_PER_WORKER, EMBED_DIM), jnp.float32),
            pltpu.VMEM((2, EMBED_DIM), jnp.float32),
            pltpu.SemaphoreType.DMA,
            pltpu.SemaphoreType.DMA,
        ],
    )(input_ids, emb_table)


def _mm_body(x_ref, w_ref, b_ref, o_ref):
    o_ref[...] = (
        lax.dot_general(
            x_ref[...],
            w_ref[...],
            (((1,), (1,)), ((), ())),
            preferred_element_type=jnp.float32,
        )
        + b_ref[...]
    )


def _tc_classify(agg, fc_w, fc_b):
    return pl.pallas_call(
        _mm_body,
        out_shape=jax.ShapeDtypeStruct((BATCH, NUM_CLASSES), jnp.float32),
    )(agg, fc_w, fc_b)


def kernel(input_ids, emb_table, fc_w, fc_b):
    agg = _sc_agg(input_ids.astype(jnp.int32), emb_table)
    return _tc_classify(agg, fc_w, fc_b.reshape(1, NUM_CLASSES))


# speedup vs baseline: 1.2596x; 1.2202x over previous
import functools

import jax
import jax.numpy as jnp
from jax import lax
from jax.experimental import pallas as pl
from jax.experimental.pallas import tpu as pltpu
from jax.experimental.pallas import tpu_sc as plsc

BATCH = 4096
SEQ = 200
EMBED_DIM = 64
NUM_CLASSES = 200
VOCAB = 1000000

NUM_CORES = 2
NUM_SUBCORES = 16
NUM_WORKERS = NUM_CORES * NUM_SUBCORES
ROWS_PER_WORKER = BATCH // NUM_WORKERS
NBUF = 2
SPLIT_A = 104
SPLIT_B = SEQ - SPLIT_A
LANES = 16
DCHUNKS = EMBED_DIM // LANES
CHUNK_OFFS = tuple(range(0, SEQ - LANES, LANES)) + (SEQ - LANES,)

DETILE_CHUNK = 2048
DETILE_GRID = -(-VOCAB // DETILE_CHUNK)
PACKED_ROWS = DETILE_GRID * (DETILE_CHUNK // 2)


def _detile_body(x_ref, o_ref):
    y = x_ref[...].T
    o_ref[...] = jnp.concatenate(
        [y[: DETILE_CHUNK // 2], y[DETILE_CHUNK // 2 :]], axis=1
    )


def _tc_detile(table_t):
    return pl.pallas_call(
        _detile_body,
        grid=(DETILE_GRID,),
        in_specs=[
            pl.BlockSpec((EMBED_DIM, DETILE_CHUNK), lambda i: (0, i)),
        ],
        out_specs=pl.BlockSpec((DETILE_CHUNK // 2, 128), lambda i: (i, 0)),
        out_shape=jax.ShapeDtypeStruct((PACKED_ROWS, 128), jnp.float32),
    )(table_t)


def _sc_body(ids_hbm, table_hbm, out_hbm, idx_v, bufs, qbufs, aggbuf, t01, sems):
    wid = lax.axis_index("s") * NUM_CORES + lax.axis_index("c")
    base = wid * ROWS_PER_WORKER

    pltpu.sync_copy(ids_hbm.at[pl.ds(base, ROWS_PER_WORKER)], idx_v)
    pltpu.sync_copy(table_hbm.at[pl.ds(0, 1)], t01.at[pl.ds(0, 1)])
    pltpu.sync_copy(table_hbm.at[pl.ds(2, 1)], t01.at[pl.ds(1, 1)])

    def prep_qidx(r, qv):
        for off in CHUNK_OFFS:
            t = idx_v[r, pl.ds(off, LANES)]
            q = (
                ((t >> 11) << 11)
                + ((t & 1023) << 1)
                + ((t >> 10) & 1)
            )
            qv[pl.ds(off, LANES)] = q

    def gather_row(buf, qv, sem):
        c0 = pltpu.make_async_copy(
            table_hbm.at[qv.at[pl.ds(0, SPLIT_A)]],
            buf.at[pl.ds(0, SPLIT_A)],
            sem,
        )
        c1 = pltpu.make_async_copy(
            table_hbm.at[qv.at[pl.ds(SPLIT_A, SPLIT_B)]],
            buf.at[pl.ds(SPLIT_A, SPLIT_B)],
            sem,
        )
        return c0, c1

    for p in range(NBUF):
        prep_qidx(p, qbufs[p])
        c0, c1 = gather_row(bufs[p], qbufs[p], sems[p])
        c0.start()
        c1.start()

    lane = lax.iota(jnp.int32, 16)
    tail_mask = lane >= (LANES - SEQ % LANES)

    def process_row(r, buf):
        n0 = jnp.zeros((LANES,), jnp.int32)
        n1 = jnp.zeros((LANES,), jnp.int32)
        for k in range(SEQ // LANES):
            v = idx_v[r, pl.ds(k * LANES, LANES)]
            n0 += plsc.all_reduce_population_count(v == 0)
            n1 += plsc.all_reduce_population_count(v == 1)
        v = idx_v[r, pl.ds(SEQ - LANES, LANES)]
        n0 += plsc.all_reduce_population_count((v == 0) & tail_mask)
        n1 += plsc.all_reduce_population_count((v == 1) & tail_mask)

        def acc_body(j, accs):
            return tuple(
                accs[c] + buf[j, pl.ds(c * LANES, LANES)] for c in range(DCHUNKS)
            )

        zero = jnp.zeros((LANES,), jnp.float32)
        accs = lax.fori_loop(0, SEQ, acc_body, (zero,) * DCHUNKS)

        n0f = n0.astype(jnp.float32)
        n1f = n1.astype(jnp.float32)
        inv = 1.0 / (float(SEQ) - n0f - n1f)
        for c in range(DCHUNKS):
            t0c = t01[0, pl.ds(c * LANES, LANES)]
            t1c = t01[1, pl.ds(c * LANES, LANES)]
            aggbuf[r, pl.ds(c * LANES, LANES)] = (
                accs[c] - n0f * t0c - n1f * t1c
            ) * inv

    def loop_body(i, carry):
        for p in range(NBUF):
            r = i * NBUF + p
            c0, c1 = gather_row(bufs[p], qbufs[p], sems[p])
            c0.wait()
            c1.wait()
            process_row(r, bufs[p])

            @pl.when(r + NBUF < ROWS_PER_WORKER)
            def _():
                prep_qidx(r + NBUF, qbufs[p])
                d0, d1 = gather_row(bufs[p], qbufs[p], sems[p])
                d0.start()
                d1.start()

        return carry

    lax.fori_loop(0, ROWS_PER_WORKER // NBUF, loop_body, 0)

    pltpu.sync_copy(aggbuf, out_hbm.at[pl.ds(base, ROWS_PER_WORKER)])


def _sc_agg(input_ids, table_lin):
    mesh = plsc.VectorSubcoreMesh(
        core_axis_name="c",
        subcore_axis_name="s",
        num_cores=NUM_CORES,
        num_subcores=NUM_SUBCORES,
    )

    def body(
        ids_hbm, table_hbm, out_hbm, idx_v, b0, b1, q0, q1, aggbuf, t01, s0, s1
    ):
        _sc_body(
            ids_hbm,
            table_hbm,
            out_hbm,
            idx_v,
            (b0, b1),
            (q0, q1),
            aggbuf,
            t01,
            (s0, s1),
        )

    return pl.kernel(
        body,
        out_type=jax.ShapeDtypeStruct((BATCH, EMBED_DIM), jnp.float32),
        mesh=mesh,
        compiler_params=pltpu.CompilerParams(
            use_tc_tiling_on_sc=False, needs_layout_passes=False
        ),
        scratch_types=[
            pltpu.VMEM((ROWS_PER_WORKER, SEQ), jnp.int32),
            pltpu.VMEM((SEQ, EMBED_DIM), jnp.float32),
            pltpu.VMEM((SEQ, EMBED_DIM), jnp.float32),
            pltpu.VMEM((SEQ,), jnp.int32),
            pltpu.VMEM((SEQ,), jnp.int32),
            pltpu.VMEM((ROWS_PER_WORKER, EMBED_DIM), jnp.float32),
            pltpu.VMEM((2, EMBED_DIM), jnp.float32),
            pltpu.SemaphoreType.DMA,
            pltpu.SemaphoreType.DMA,
        ],
    )(input_ids, table_lin)


def _mm_body(x_ref, w_ref, b_ref, o_ref):
    o_ref[...] = (
        lax.dot_general(
            x_ref[...],
            w_ref[...],
            (((1,), (1,)), ((), ())),
            preferred_element_type=jnp.float32,
        )
        + b_ref[...]
    )


def _tc_classify(agg, fc_w, fc_b):
    return pl.pallas_call(
        _mm_body,
        out_shape=jax.ShapeDtypeStruct((BATCH, NUM_CLASSES), jnp.float32),
    )(agg, fc_w, fc_b)


def kernel(input_ids, emb_table, fc_w, fc_b):
    tbl_packed = _tc_detile(emb_table.T)
    tbl_lin = tbl_packed.reshape(2 * PACKED_ROWS, EMBED_DIM)
    agg = _sc_agg(input_ids.astype(jnp.int32), tbl_lin)
    return _tc_classify(agg, fc_w, fc_b.reshape(1, NUM_CLASSES))


# speedup vs baseline: 1.8218x; 1.4464x over previous
import functools

import jax
import jax.numpy as jnp
from jax import lax
from jax.experimental import pallas as pl
from jax.experimental.pallas import tpu as pltpu
from jax.experimental.pallas import tpu_sc as plsc

BATCH = 4096
SEQ = 200
EMBED_DIM = 64
NUM_CLASSES = 200
VOCAB = 1000000

NUM_CORES = 2
NUM_SUBCORES = 16
NUM_WORKERS = NUM_CORES * NUM_SUBCORES
ROWS_PER_WORKER = BATCH // NUM_WORKERS
NBUF = 2
SPLIT_A = 104
SPLIT_B = SEQ - SPLIT_A
LANES = 16
DCHUNKS = EMBED_DIM // LANES
CHUNK_OFFS = tuple(range(0, SEQ - LANES, LANES)) + (SEQ - LANES,)

DETILE_CHUNK = 8192
DETILE_LOG_C = 13
DETILE_LOG_H = 12
DETILE_GRID = -(-VOCAB // DETILE_CHUNK)
PACKED_ROWS = DETILE_GRID * (DETILE_CHUNK // 2)


def _detile_body(x_ref, i_ref, o_ref):
    y = lax.dot_general(
        x_ref[...],
        i_ref[...],
        (((0,), (0,)), ((), ())),
        preferred_element_type=jnp.float32,
    )
    o_ref[...] = jnp.concatenate(
        [y[: DETILE_CHUNK // 2], y[DETILE_CHUNK // 2 :]], axis=1
    )


def _tc_detile(table_t):
    ident = jnp.eye(EMBED_DIM, dtype=jnp.float32)
    return pl.pallas_call(
        _detile_body,
        grid=(DETILE_GRID,),
        in_specs=[
            pl.BlockSpec((EMBED_DIM, DETILE_CHUNK), lambda i: (0, i)),
            pl.BlockSpec((EMBED_DIM, EMBED_DIM), lambda i: (0, 0)),
        ],
        out_specs=pl.BlockSpec((DETILE_CHUNK // 2, 128), lambda i: (i, 0)),
        out_shape=jax.ShapeDtypeStruct((PACKED_ROWS, 128), jnp.float32),
    )(table_t, ident)


def _sc_body(ids_hbm, table_hbm, out_hbm, idx_v, bufs, qbufs, aggbuf, t01, sems):
    wid = lax.axis_index("s") * NUM_CORES + lax.axis_index("c")
    base = wid * ROWS_PER_WORKER

    pltpu.sync_copy(ids_hbm.at[pl.ds(base, ROWS_PER_WORKER)], idx_v)
    pltpu.sync_copy(table_hbm.at[pl.ds(0, 1)], t01.at[pl.ds(0, 1)])
    pltpu.sync_copy(table_hbm.at[pl.ds(2, 1)], t01.at[pl.ds(1, 1)])

    def prep_qidx(r, qv):
        for off in CHUNK_OFFS:
            t = idx_v[r, pl.ds(off, LANES)]
            q = (
                ((t >> DETILE_LOG_C) << DETILE_LOG_C)
                + ((t & (DETILE_CHUNK // 2 - 1)) << 1)
                + ((t >> DETILE_LOG_H) & 1)
            )
            qv[pl.ds(off, LANES)] = q

    def gather_row(buf, qv, sem):
        c0 = pltpu.make_async_copy(
            table_hbm.at[qv.at[pl.ds(0, SPLIT_A)]],
            buf.at[pl.ds(0, SPLIT_A)],
            sem,
        )
        c1 = pltpu.make_async_copy(
            table_hbm.at[qv.at[pl.ds(SPLIT_A, SPLIT_B)]],
            buf.at[pl.ds(SPLIT_A, SPLIT_B)],
            sem,
        )
        return c0, c1

    for p in range(NBUF):
        prep_qidx(p, qbufs[p])
        c0, c1 = gather_row(bufs[p], qbufs[p], sems[p])
        c0.start()
        c1.start()

    lane = lax.iota(jnp.int32, 16)
    tail_mask = lane >= (LANES - SEQ % LANES)

    def process_row(r, buf):
        n0 = jnp.zeros((LANES,), jnp.int32)
        n1 = jnp.zeros((LANES,), jnp.int32)
        for k in range(SEQ // LANES):
            v = idx_v[r, pl.ds(k * LANES, LANES)]
            n0 += plsc.all_reduce_population_count(v == 0)
            n1 += plsc.all_reduce_population_count(v == 1)
        v = idx_v[r, pl.ds(SEQ - LANES, LANES)]
        n0 += plsc.all_reduce_population_count((v == 0) & tail_mask)
        n1 += plsc.all_reduce_population_count((v == 1) & tail_mask)

        def acc_body(j, accs):
            return tuple(
                accs[c] + buf[j, pl.ds(c * LANES, LANES)] for c in range(DCHUNKS)
            )

        zero = jnp.zeros((LANES,), jnp.float32)
        accs = lax.fori_loop(0, SEQ, acc_body, (zero,) * DCHUNKS)

        n0f = n0.astype(jnp.float32)
        n1f = n1.astype(jnp.float32)
        inv = 1.0 / (float(SEQ) - n0f - n1f)
        for c in range(DCHUNKS):
            t0c = t01[0, pl.ds(c * LANES, LANES)]
            t1c = t01[1, pl.ds(c * LANES, LANES)]
            aggbuf[r, pl.ds(c * LANES, LANES)] = (
                accs[c] - n0f * t0c - n1f * t1c
            ) * inv

    def loop_body(i, carry):
        for p in range(NBUF):
            r = i * NBUF + p
            c0, c1 = gather_row(bufs[p], qbufs[p], sems[p])
            c0.wait()
            c1.wait()
            process_row(r, bufs[p])

            @pl.when(r + NBUF < ROWS_PER_WORKER)
            def _():
                prep_qidx(r + NBUF, qbufs[p])
                d0, d1 = gather_row(bufs[p], qbufs[p], sems[p])
                d0.start()
                d1.start()

        return carry

    lax.fori_loop(0, ROWS_PER_WORKER // NBUF, loop_body, 0)

    pltpu.sync_copy(aggbuf, out_hbm.at[pl.ds(base, ROWS_PER_WORKER)])


def _sc_agg(input_ids, table_lin):
    mesh = plsc.VectorSubcoreMesh(
        core_axis_name="c",
        subcore_axis_name="s",
        num_cores=NUM_CORES,
        num_subcores=NUM_SUBCORES,
    )

    def body(
        ids_hbm, table_hbm, out_hbm, idx_v, b0, b1, q0, q1, aggbuf, t01, s0, s1
    ):
        _sc_body(
            ids_hbm,
            table_hbm,
            out_hbm,
            idx_v,
            (b0, b1),
            (q0, q1),
            aggbuf,
            t01,
            (s0, s1),
        )

    return pl.kernel(
        body,
        out_type=jax.ShapeDtypeStruct((BATCH, EMBED_DIM), jnp.float32),
        mesh=mesh,
        compiler_params=pltpu.CompilerParams(
            use_tc_tiling_on_sc=False, needs_layout_passes=False
        ),
        scratch_types=[
            pltpu.VMEM((ROWS_PER_WORKER, SEQ), jnp.int32),
            pltpu.VMEM((SEQ, EMBED_DIM), jnp.float32),
            pltpu.VMEM((SEQ, EMBED_DIM), jnp.float32),
            pltpu.VMEM((SEQ,), jnp.int32),
            pltpu.VMEM((SEQ,), jnp.int32),
            pltpu.VMEM((ROWS_PER_WORKER, EMBED_DIM), jnp.float32),
            pltpu.VMEM((2, EMBED_DIM), jnp.float32),
            pltpu.SemaphoreType.DMA,
            pltpu.SemaphoreType.DMA,
        ],
    )(input_ids, table_lin)


def _mm_body(x_ref, w_ref, b_ref, o_ref):
    o_ref[...] = (
        lax.dot_general(
            x_ref[...],
            w_ref[...],
            (((1,), (1,)), ((), ())),
            preferred_element_type=jnp.float32,
        )
        + b_ref[...]
    )


def _tc_classify(agg, fc_w, fc_b):
    return pl.pallas_call(
        _mm_body,
        out_shape=jax.ShapeDtypeStruct((BATCH, NUM_CLASSES), jnp.float32),
    )(agg, fc_w, fc_b)


def kernel(input_ids, emb_table, fc_w, fc_b):
    tbl_packed = _tc_detile(emb_table.T)
    tbl_lin = tbl_packed.reshape(2 * PACKED_ROWS, EMBED_DIM)
    agg = _sc_agg(input_ids.astype(jnp.int32), tbl_lin)
    return _tc_classify(agg, fc_w, fc_b.reshape(1, NUM_CLASSES))
